# baseline (speedup 1.0000x reference)
import functools

import jax
import jax.numpy as jnp
from jax import lax
from jax.experimental import pallas as pl
from jax.experimental.pallas import tpu as pltpu
from jax.experimental.pallas import tpu_sc as plsc

_NUM_PROJS = 16
_HPB = 4


def _pack_body(mat_ref, proj_ref, out_ref):
    x = mat_ref[0]
    p = proj_ref[0]
    y = jax.lax.dot_general(
        p, x, (((1,), (2,)), ((0,), (0,))),
        preferred_element_type=jnp.float32,
    )
    bits = (y > 0).astype(jnp.int32)
    enc = jnp.left_shift(
        jnp.int32(1),
        jax.lax.broadcasted_iota(jnp.int32, (1, _NUM_PROJS, 1), 1),
    )
    out_ref[0] = jnp.sum(bits * enc, axis=1)


def _tc_bucket_ids(mat, proj_dir):
    B, H, S, D = mat.shape
    grid = (B * H) // _HPB
    n_proj_grp = H // _HPB
    mat_r = mat.reshape(grid, _HPB, S, D)
    proj_r = proj_dir.reshape(n_proj_grp, _HPB, D, _NUM_PROJS)
    out = pl.pallas_call(
        _pack_body,
        grid=(grid,),
        in_specs=[
            pl.BlockSpec((1, _HPB, S, D), lambda i: (i, 0, 0, 0)),
            pl.BlockSpec((1, _HPB, D, _NUM_PROJS),
                         lambda i: (i % n_proj_grp, 0, 0, 0)),
        ],
        out_specs=pl.BlockSpec((1, _HPB, S), lambda i: (i, 0, 0)),
        out_shape=jax.ShapeDtypeStruct((grid, _HPB, S), jnp.int32),
    )(mat_r, proj_r)
    return out.reshape(B * H * S)


def _sc_perm_gather(table_hi, table_lo, idx):
    n = idx.shape[0]
    info = plsc.get_sparse_core_info()
    nw = info.num_cores * info.num_subcores
    nl = info.num_lanes
    bpw = n // nw
    mesh = plsc.VectorSubcoreMesh(core_axis_name="c", subcore_axis_name="s")

    @functools.partial(
        pl.kernel,
        out_type=jax.ShapeDtypeStruct((n,), jnp.int32),
        mesh=mesh,
        compiler_params=pltpu.CompilerParams(needs_layout_passes=False),
        scratch_types=[
            pltpu.VMEM((table_hi.shape[0],), jnp.int32),
            pltpu.VMEM((table_lo.shape[0],), jnp.int32),
            pltpu.VMEM((bpw,), jnp.int32),
            pltpu.VMEM((bpw,), jnp.int32),
            pltpu.SemaphoreType.DMA,
        ],
    )
    def k(thi_hbm, tlo_hbm, idx_hbm, out_hbm, thi_v, tlo_v, idx_v, rows_v,
          sem):
        wid = lax.axis_index("s") * info.num_cores + lax.axis_index("c")
        base = wid * bpw
        cp_hi = pltpu.async_copy(thi_hbm, thi_v, sem)
        cp_lo = pltpu.async_copy(tlo_hbm, tlo_v, sem)
        pltpu.sync_copy(idx_hbm.at[pl.ds(base, bpw)], idx_v)
        cp_hi.wait()
        cp_lo.wait()

        def body(j, carry):
            iv = idx_v[pl.ds(j * nl, nl)]
            hi = plsc.load_gather(thi_v, [jnp.right_shift(iv, 8)])
            lo = plsc.load_gather(tlo_v, [jnp.bitwise_and(iv, 0x1FF)])
            rows_v[pl.ds(j * nl, nl)] = jnp.bitwise_or(hi, lo)
            return carry

        lax.fori_loop(0, bpw // nl, body, 0)
        pltpu.sync_copy(rows_v, out_hbm.at[pl.ds(base, bpw)])

    return k(table_hi, table_lo, idx)


def kernel(mat, proj_dir):
    B, H, S, _ = mat.shape
    bin_ids = _tc_bucket_ids(mat, proj_dir)
    i8 = jnp.arange(256, dtype=jnp.int32)
    table_hi = jnp.left_shift(i8 ^ (i8 >> 1), 8)
    i9 = jnp.arange(512, dtype=jnp.int32)
    table_lo = jnp.bitwise_and(i9 ^ (i9 >> 1), 0xFF)
    out = _sc_perm_gather(table_hi, table_lo, bin_ids)
    return out.reshape(B, H, S)


# speedup vs baseline: 1.0161x; 1.0161x over previous
import functools

import jax
import jax.numpy as jnp
from jax import lax
from jax.experimental import pallas as pl
from jax.experimental.pallas import tpu as pltpu
from jax.experimental.pallas import tpu_sc as plsc

_NUM_PROJS = 16
_HPB = 4


def _pack_body(mat_ref, proj_ref, out_ref):
    x = mat_ref[0]
    p = proj_ref[0]
    y = jax.lax.dot_general(
        p, x, (((1,), (2,)), ((0,), (0,))),
        preferred_element_type=jnp.float32,
    )
    bits = (y > 0).astype(jnp.int32)
    enc = jnp.left_shift(
        jnp.int32(1),
        jax.lax.broadcasted_iota(jnp.int32, (1, _NUM_PROJS, 1), 1),
    )
    out_ref[0] = jnp.sum(bits * enc, axis=1)


def _tc_bucket_ids(mat, proj_dir):
    B, H, S, D = mat.shape
    grid = (B * H) // _HPB
    n_proj_grp = H // _HPB
    mat_r = mat.reshape(grid, _HPB, S, D)
    proj_r = proj_dir.reshape(n_proj_grp, _HPB, D, _NUM_PROJS)
    out = pl.pallas_call(
        _pack_body,
        grid=(grid,),
        in_specs=[
            pl.BlockSpec((1, _HPB, S, D), lambda i: (i, 0, 0, 0)),
            pl.BlockSpec((1, _HPB, D, _NUM_PROJS),
                         lambda i: (i % n_proj_grp, 0, 0, 0)),
        ],
        out_specs=pl.BlockSpec((1, _HPB, S), lambda i: (i, 0, 0)),
        out_shape=jax.ShapeDtypeStruct((grid, _HPB, S), jnp.int32),
    )(mat_r, proj_r)
    return out.reshape(B * H * S)


def _sc_perm_gather(table_hi, table_lo, idx):
    n = idx.shape[0]
    info = plsc.get_sparse_core_info()
    nw = info.num_cores * info.num_subcores
    nl = info.num_lanes
    bpw = n // nw
    mesh = plsc.VectorSubcoreMesh(core_axis_name="c", subcore_axis_name="s")

    @functools.partial(
        pl.kernel,
        out_type=jax.ShapeDtypeStruct((n,), jnp.int32),
        mesh=mesh,
        compiler_params=pltpu.CompilerParams(needs_layout_passes=False),
        scratch_types=[
            pltpu.VMEM((table_hi.shape[0],), jnp.int32),
            pltpu.VMEM((table_lo.shape[0],), jnp.int32),
            pltpu.VMEM((bpw,), jnp.int32),
            pltpu.VMEM((bpw,), jnp.int32),
            pltpu.SemaphoreType.DMA,
        ],
    )
    def k(thi_hbm, tlo_hbm, idx_hbm, out_hbm, thi_v, tlo_v, idx_v, rows_v,
          sem):
        wid = lax.axis_index("s") * info.num_cores + lax.axis_index("c")
        base = wid * bpw
        cp_hi = pltpu.async_copy(thi_hbm, thi_v, sem)
        cp_lo = pltpu.async_copy(tlo_hbm, tlo_v, sem)
        pltpu.sync_copy(idx_hbm.at[pl.ds(base, bpw)], idx_v)
        cp_hi.wait()
        cp_lo.wait()

        unroll = 4

        def body(j, carry):
            for u in range(unroll):
                off = (j * unroll + u) * nl
                iv = idx_v[pl.ds(off, nl)]
                hi = plsc.load_gather(thi_v, [jnp.right_shift(iv, 8)])
                lo = plsc.load_gather(tlo_v, [jnp.bitwise_and(iv, 0x1FF)])
                rows_v[pl.ds(off, nl)] = jnp.bitwise_or(hi, lo)
            return carry

        lax.fori_loop(0, bpw // (nl * unroll), body, 0)
        pltpu.sync_copy(rows_v, out_hbm.at[pl.ds(base, bpw)])

    return k(table_hi, table_lo, idx)


def kernel(mat, proj_dir):
    B, H, S, _ = mat.shape
    bin_ids = _tc_bucket_ids(mat, proj_dir)
    i8 = jnp.arange(256, dtype=jnp.int32)
    table_hi = jnp.left_shift(i8 ^ (i8 >> 1), 8)
    i9 = jnp.arange(512, dtype=jnp.int32)
    table_lo = jnp.bitwise_and(i9 ^ (i9 >> 1), 0xFF)
    out = _sc_perm_gather(table_hi, table_lo, bin_ids)
    return out.reshape(B, H, S)
